# Initial kernel scaffold; baseline (speedup 1.0000x reference)
import jax
import jax.numpy as jnp
from jax.experimental import pallas as pl


def kernel(weight, edge_index, edge_weight):
    raise NotImplementedError("write your pallas kernel here")


# speedup vs baseline: 2.9873x; 2.9873x over previous
import functools

import jax
import jax.numpy as jnp
from jax import lax
from jax.experimental import pallas as pl
from jax.experimental.pallas import tpu as pltpu
from jax.experimental.pallas import tpu_sc as plsc

N_NODES = 10000
N_EDGES = 160000
DIM = 256
HALF = DIM // 2
NUM_HOPS = 3
MIN_NORM = 1e-15
EPS = 1e-7

NC = 2
NS = 16
LANES = 16

EPT = N_EDGES // NS
CHUNK = 128
NFULL = EPT // CHUNK
REM = EPT - NFULL * CHUNK
N_PAD = 10240
RPT = N_PAD // NS
ZCH = 128


def _pre_body(w_ref, o_ref):
    w = w_ref[...]
    d = w[:, 1:]
    y2 = jnp.sum(d * d, axis=1, keepdims=True)
    x0 = jnp.sqrt(jnp.clip(1.0 + y2, EPS, None))
    y_norm = jnp.clip(jnp.sqrt(y2), MIN_NORM, None)
    theta = jnp.clip(x0, 1.0 + EPS, None)
    r = jnp.log(theta + jnp.sqrt(theta * theta - 1.0))
    res = (r / y_norm) * d
    xt = jnp.concatenate([jnp.zeros_like(w[:, :1]), res], axis=1)
    o_ref[0] = xt[:, :HALF]
    o_ref[1] = xt[:, HALF:]


def _pre(weight):
    rows = 1000
    return pl.pallas_call(
        _pre_body,
        grid=(N_NODES // rows,),
        in_specs=[pl.BlockSpec((rows, DIM), lambda i: (i, 0))],
        out_specs=pl.BlockSpec((2, rows, HALF), lambda i: (0, i, 0)),
        out_shape=jax.ShapeDtypeStruct((2, N_NODES, HALF), jnp.float32),
    )(weight)


def _post_body(h_ref, o_ref):
    h = h_ref[...]
    acc = h[0] + h[1] + h[2]
    u = jnp.concatenate([acc[0], acc[1]], axis=1)
    d = u[:, 1:]
    x_norm = jnp.clip(jnp.sqrt(jnp.sum(d * d, axis=1, keepdims=True)),
                      MIN_NORM, None)
    sinh = 0.5 * (jnp.exp(x_norm) - jnp.exp(-x_norm))
    rest = sinh * d / x_norm
    y2 = jnp.sum(rest * rest, axis=1, keepdims=True)
    x0 = jnp.sqrt(jnp.clip(1.0 + y2, EPS, None))
    o_ref[...] = jnp.concatenate([x0, rest], axis=1)


def _post(hs):
    rows = 1000
    return pl.pallas_call(
        _post_body,
        grid=(N_NODES // rows,),
        in_specs=[pl.BlockSpec((NUM_HOPS, 2, rows, HALF),
                               lambda i: (0, 0, i, 0))],
        out_specs=pl.BlockSpec((rows, DIM), lambda i: (i, 0)),
        out_shape=jax.ShapeDtypeStruct((N_NODES, DIM), jnp.float32),
    )(hs)


def _sc_body(h0, srcs, dsts, ws, out,
             src_v, dst_v, w_v, rows_v, src_r, dst_r, w_r, rows_r,
             acc_sh, sem):
    c = lax.axis_index("c")
    s = lax.axis_index("s")
    ebase = s * EPT
    zeros16 = jnp.zeros((LANES,), jnp.float32)

    def zero_rows(n):
        def body(e, _):
            for j in range(HALF // LANES):
                rows_v[e, pl.ds(j * LANES, LANES)] = zeros16
            return 0
        lax.fori_loop(0, n, body, 0)

    for hop in range(NUM_HOPS):
        zero_rows(ZCH)
        for z in range(RPT // ZCH):
            base = s * RPT + z * ZCH
            pltpu.sync_copy(rows_v.at[pl.ds(0, ZCH)],
                            acc_sh.at[pl.ds(base, ZCH)])
        plsc.subcore_barrier()

        def chunk(off, si, di, wi, rv, n):
            pltpu.sync_copy(srcs.at[pl.ds(off, n)], si)
            pltpu.sync_copy(dsts.at[pl.ds(off, n)], di)
            pltpu.sync_copy(ws.at[pl.ds(off, n)], wi.at[pl.ds(0, n)])
            if hop == 0:
                pltpu.async_copy(h0.at[c].at[si], rv, sem).wait()
            else:
                pltpu.async_copy(out.at[hop - 1, c].at[si], rv, sem).wait()

            def sbody(e, _):
                w = wi[pl.ds(e, LANES)][0]
                for j in range(HALF // LANES):
                    sl = rv[e, pl.ds(j * LANES, LANES)]
                    rv[e, pl.ds(j * LANES, LANES)] = sl * w
                return 0
            lax.fori_loop(0, n, sbody, 0)
            pltpu.sync_copy(rv, acc_sh.at[di], add=True)

        def chunk_body(i, _):
            chunk(ebase + i * CHUNK, src_v, dst_v, w_v, rows_v, CHUNK)
            return 0
        lax.fori_loop(0, NFULL, chunk_body, 0)
        chunk(ebase + NFULL * CHUNK, src_r, dst_r, w_r, rows_r, REM)
        plsc.subcore_barrier()

        pltpu.sync_copy(acc_sh.at[pl.ds(s * RPT, RPT)],
                        out.at[hop, c, pl.ds(s * RPT, RPT)])
        plsc.subcore_barrier()


def _spmm(xt2, srcs, dsts, ws):
    mesh = plsc.VectorSubcoreMesh(core_axis_name="c", subcore_axis_name="s")
    f = functools.partial(
        pl.kernel,
        mesh=mesh,
        out_type=jax.ShapeDtypeStruct((NUM_HOPS, 2, N_PAD, HALF),
                                      jnp.float32),
        scratch_types=[
            pltpu.VMEM((CHUNK,), jnp.int32),
            pltpu.VMEM((CHUNK,), jnp.int32),
            pltpu.VMEM((CHUNK + LANES,), jnp.float32),
            pltpu.VMEM((CHUNK, HALF), jnp.float32),
            pltpu.VMEM((REM,), jnp.int32),
            pltpu.VMEM((REM,), jnp.int32),
            pltpu.VMEM((REM + LANES,), jnp.float32),
            pltpu.VMEM((REM, HALF), jnp.float32),
            pltpu.VMEM_SHARED((N_PAD, HALF), jnp.float32),
            pltpu.SemaphoreType.DMA,
        ],
    )(_sc_body)
    return f(xt2, srcs, dsts, ws)


def kernel(weight, edge_index, edge_weight):
    xt2 = _pre(weight)
    srcs = edge_index[0]
    dsts = edge_index[1]
    hs = _spmm(xt2, srcs, dsts, edge_weight)
    return _post(hs)
